# baseline (speedup 1.0000x reference)
import functools

import jax
import jax.numpy as jnp
from jax.experimental import pallas as pl

_TAU = 0.05


def _feats_body(haug_ref, oaug_ref, h_ref, o_ref, shc_ref, f_ref, *, nh, no, kqs):
    haug = haug_ref[0]
    oaug = oaug_ref[0]
    h = h_ref[0]
    o = o_ref[0]
    shc = shc_ref[0]

    sq = jnp.dot(haug, oaug, preferred_element_type=jnp.float32)

    sqmin_h = jnp.min(sq, axis=1, keepdims=True)
    sqmin_o = jnp.min(sq, axis=0, keepdims=True)
    dmin_h = jnp.sqrt(jnp.maximum(sqmin_h, 1e-12))
    dmin_o = jnp.sqrt(jnp.maximum(sqmin_o, 1e-12))

    iota_m = jax.lax.broadcasted_iota(jnp.int32, (nh, no), 1)
    idx = jnp.min(jnp.where(sq == sqmin_h, iota_m, no), axis=1, keepdims=True)
    onehot = (iota_m == idx).astype(jnp.float32)
    o_nn = jnp.dot(onehot, o, preferred_element_type=jnp.float32)

    vec = o_nn - h
    nrm = jnp.sqrt(jnp.maximum(jnp.sum(vec * vec, axis=1, keepdims=True), 1e-6))
    dir_mean = jnp.sum(vec / nrm, axis=0, keepdims=True) * (1.0 / nh)

    w_h = jnp.exp(dmin_h * (-1.0 / _TAU)) * shc

    dm_row = jnp.transpose(dmin_h)
    iota_r = jax.lax.broadcasted_iota(jnp.int32, (nh, nh), 0)
    iota_c = jax.lax.broadcasted_iota(jnp.int32, (nh, nh), 1)
    smaller = (dm_row < dmin_h) | ((dm_row == dmin_h) & (iota_c < iota_r))
    rank = jnp.dot(smaller.astype(jnp.float32),
                   jnp.ones((nh, 1), jnp.float32),
                   preferred_element_type=jnp.float32)

    qmeans = []
    for kq in kqs:
        sel = jnp.where(rank < kq, dmin_h, 0.0)
        qmeans.append(jnp.sum(sel, axis=0, keepdims=True) * (1.0 / kq))

    mean_dh = jnp.sum(dmin_h, axis=0, keepdims=True) * (1.0 / nh)
    min_dh = jnp.min(dmin_h, axis=0, keepdims=True)
    mean_wh = jnp.sum(w_h, axis=0, keepdims=True) * (1.0 / nh)
    mean_do = jnp.sum(dmin_o, axis=1, keepdims=True) * (1.0 / no)

    f_ref[...] = jnp.concatenate(
        [mean_dh, min_dh, qmeans[0], qmeans[1], qmeans[2],
         mean_wh, dir_mean, mean_do], axis=1)[None]


def _mlp_body(f_ref, w1_ref, b1_ref, w2_ref, b2_ref, out_ref):
    hid = jnp.maximum(
        jnp.dot(f_ref[...], w1_ref[...], preferred_element_type=jnp.float32)
        + b1_ref[...], 0.0)
    out_ref[...] = (
        jnp.dot(hid, w2_ref[...], preferred_element_type=jnp.float32)
        + b2_ref[...])


def kernel(human_bt_n3, object_bt_m3, s_h_bt_n, s_o_bt_m, W1, b1, W2, b2):
    B, T, Nh, _ = human_bt_n3.shape
    No = object_bt_m3.shape[2]
    BT = B * T
    h = human_bt_n3.reshape(BT, Nh, 3)
    o = object_bt_m3.reshape(BT, No, 3)
    oT = o.transpose(0, 2, 1)
    shc = s_h_bt_n.reshape(BT, Nh, 1)
    hn2 = jnp.sum(h * h, axis=2, keepdims=True)
    on2 = jnp.sum(o * o, axis=2)[:, None, :]
    ones_h = jnp.ones((BT, Nh, 1), jnp.float32)
    ones_o = jnp.ones((BT, 1, No), jnp.float32)
    haug = jnp.concatenate([-2.0 * h, hn2, ones_h], axis=2)
    oaug = jnp.concatenate([oT, ones_o, on2], axis=1)
    kqs = tuple(int(max(1, round(q * Nh))) for q in (0.2, 0.5, 0.8))

    feats = pl.pallas_call(
        functools.partial(_feats_body, nh=Nh, no=No, kqs=kqs),
        grid=(BT,),
        in_specs=[
            pl.BlockSpec((1, Nh, 5), lambda i: (i, 0, 0)),
            pl.BlockSpec((1, 5, No), lambda i: (i, 0, 0)),
            pl.BlockSpec((1, Nh, 3), lambda i: (i, 0, 0)),
            pl.BlockSpec((1, No, 3), lambda i: (i, 0, 0)),
            pl.BlockSpec((1, Nh, 1), lambda i: (i, 0, 0)),
        ],
        out_specs=pl.BlockSpec((1, 1, 10), lambda i: (i, 0, 0)),
        out_shape=jax.ShapeDtypeStruct((BT, 1, 10), jnp.float32),
    )(haug, oaug, h, o, shc)
    feats = feats.reshape(BT, 10)

    H = W1.shape[1]
    F = W2.shape[1]
    out = pl.pallas_call(
        _mlp_body,
        in_specs=[pl.BlockSpec(feats.shape, lambda: (0, 0)),
                  pl.BlockSpec(W1.shape, lambda: (0, 0)),
                  pl.BlockSpec((1, H), lambda: (0, 0)),
                  pl.BlockSpec(W2.shape, lambda: (0, 0)),
                  pl.BlockSpec((1, F), lambda: (0, 0))],
        out_specs=pl.BlockSpec((BT, F), lambda: (0, 0)),
        out_shape=jax.ShapeDtypeStruct((BT, F), jnp.float32),
    )(feats, W1, b1.reshape(1, H), W2, b2.reshape(1, F))
    return out.reshape(B, T, F)


# speedup vs baseline: 1.6251x; 1.6251x over previous
import functools

import jax
import jax.numpy as jnp
from jax.experimental import pallas as pl

_TAU = 0.05


def _feats_body(h_ref, o_ref, oT_ref, shc_ref, f_ref, *, nh, no, kqs):
    h = h_ref[0]
    o = o_ref[0]
    oT = oT_ref[0]
    shc = shc_ref[0]

    a2 = jnp.sum(h * h, axis=1, keepdims=True)
    b2 = jnp.sum(oT * oT, axis=0, keepdims=True)
    g = jnp.dot(h, oT, preferred_element_type=jnp.float32)
    sq = a2 + b2 - 2.0 * g

    sqmin_h = jnp.min(sq, axis=1, keepdims=True)
    sqmin_o = jnp.min(sq, axis=0, keepdims=True)
    dmin_h = jnp.sqrt(jnp.maximum(sqmin_h, 1e-12))
    dmin_o = jnp.sqrt(jnp.maximum(sqmin_o, 1e-12))

    iota_m = jax.lax.broadcasted_iota(jnp.int32, (nh, no), 1)
    idx = jnp.min(jnp.where(sq == sqmin_h, iota_m, no), axis=1, keepdims=True)
    onehot = (iota_m == idx).astype(jnp.float32)
    o_nn = jnp.dot(onehot, o, preferred_element_type=jnp.float32)

    vec = o_nn - h
    nrm = jnp.sqrt(jnp.maximum(jnp.sum(vec * vec, axis=1, keepdims=True), 1e-6))
    dir_mean = jnp.sum(vec / nrm, axis=0, keepdims=True) * (1.0 / nh)

    w_h = jnp.exp(dmin_h * (-1.0 / _TAU)) * shc

    dm_row = jnp.transpose(dmin_h)
    iota_r = jax.lax.broadcasted_iota(jnp.int32, (nh, nh), 0)
    iota_c = jax.lax.broadcasted_iota(jnp.int32, (nh, nh), 1)
    smaller = (dm_row < dmin_h) | ((dm_row == dmin_h) & (iota_c < iota_r))
    rank = jnp.dot(smaller.astype(jnp.float32),
                   jnp.ones((nh, 1), jnp.float32),
                   preferred_element_type=jnp.float32)

    qmeans = []
    for kq in kqs:
        sel = jnp.where(rank < kq, dmin_h, 0.0)
        qmeans.append(jnp.sum(sel, axis=0, keepdims=True) * (1.0 / kq))

    mean_dh = jnp.sum(dmin_h, axis=0, keepdims=True) * (1.0 / nh)
    min_dh = jnp.min(dmin_h, axis=0, keepdims=True)
    mean_wh = jnp.sum(w_h, axis=0, keepdims=True) * (1.0 / nh)
    mean_do = jnp.sum(dmin_o, axis=1, keepdims=True) * (1.0 / no)

    f_ref[...] = jnp.concatenate(
        [mean_dh, min_dh, qmeans[0], qmeans[1], qmeans[2],
         mean_wh, dir_mean, mean_do], axis=1)[None]


def _mlp_body(f_ref, w1_ref, b1_ref, w2_ref, b2_ref, out_ref):
    hid = jnp.maximum(
        jnp.dot(f_ref[...], w1_ref[...], preferred_element_type=jnp.float32)
        + b1_ref[...], 0.0)
    out_ref[...] = (
        jnp.dot(hid, w2_ref[...], preferred_element_type=jnp.float32)
        + b2_ref[...])


def kernel(human_bt_n3, object_bt_m3, s_h_bt_n, s_o_bt_m, W1, b1, W2, b2):
    B, T, Nh, _ = human_bt_n3.shape
    No = object_bt_m3.shape[2]
    BT = B * T
    h = human_bt_n3.reshape(BT, Nh, 3)
    o = object_bt_m3.reshape(BT, No, 3)
    oT = o.transpose(0, 2, 1)
    shc = s_h_bt_n.reshape(BT, Nh, 1)
    kqs = tuple(int(max(1, round(q * Nh))) for q in (0.2, 0.5, 0.8))

    feats = pl.pallas_call(
        functools.partial(_feats_body, nh=Nh, no=No, kqs=kqs),
        grid=(BT,),
        in_specs=[
            pl.BlockSpec((1, Nh, 3), lambda i: (i, 0, 0)),
            pl.BlockSpec((1, No, 3), lambda i: (i, 0, 0)),
            pl.BlockSpec((1, 3, No), lambda i: (i, 0, 0)),
            pl.BlockSpec((1, Nh, 1), lambda i: (i, 0, 0)),
        ],
        out_specs=pl.BlockSpec((1, 1, 10), lambda i: (i, 0, 0)),
        out_shape=jax.ShapeDtypeStruct((BT, 1, 10), jnp.float32),
    )(h, o, oT, shc)
    feats = feats.reshape(BT, 10)

    H = W1.shape[1]
    F = W2.shape[1]
    out = pl.pallas_call(
        _mlp_body,
        in_specs=[pl.BlockSpec(feats.shape, lambda: (0, 0)),
                  pl.BlockSpec(W1.shape, lambda: (0, 0)),
                  pl.BlockSpec((1, H), lambda: (0, 0)),
                  pl.BlockSpec(W2.shape, lambda: (0, 0)),
                  pl.BlockSpec((1, F), lambda: (0, 0))],
        out_specs=pl.BlockSpec((BT, F), lambda: (0, 0)),
        out_shape=jax.ShapeDtypeStruct((BT, F), jnp.float32),
    )(feats, W1, b1.reshape(1, H), W2, b2.reshape(1, F))
    return out.reshape(B, T, F)


# speedup vs baseline: 1.9731x; 1.2142x over previous
import functools

import jax
import jax.numpy as jnp
from jax.experimental import pallas as pl

_TAU = 0.05


def _feats_body(o_ref, hT_ref, oT_ref, sh_ref, f_ref, *, nh, no, kqs):
    o = o_ref[0]
    hT = hT_ref[0]
    oT = oT_ref[0]
    shr = sh_ref[0]

    a2r = jnp.sum(hT * hT, axis=0, keepdims=True)
    b2r = jnp.sum(oT * oT, axis=0, keepdims=True)
    b2c = jnp.transpose(b2r)
    gT = jnp.dot(o, hT, preferred_element_type=jnp.float32)
    sqT = b2c + a2r - 2.0 * gT

    sqmin_h = jnp.min(sqT, axis=0, keepdims=True)
    sqmin_o = jnp.min(sqT, axis=1, keepdims=True)
    dmin_h = jnp.sqrt(jnp.maximum(sqmin_h, 1e-12))
    dmin_o = jnp.sqrt(jnp.maximum(jnp.transpose(sqmin_o), 1e-12))

    eqf = (sqT == sqmin_h).astype(jnp.float32)
    o_nn = jnp.dot(oT, eqf, preferred_element_type=jnp.float32)
    cnt = jnp.dot(jnp.ones((1, no), jnp.float32), eqf,
                  preferred_element_type=jnp.float32)

    vecT = o_nn / cnt - hT
    nrm = jnp.sqrt(jnp.maximum(
        jnp.sum(vecT * vecT, axis=0, keepdims=True), 1e-6))
    dir_sum = jnp.sum(vecT / nrm, axis=1, keepdims=True)
    dir_mean = jnp.transpose(dir_sum) * (1.0 / nh)

    w_h = jnp.exp(dmin_h * (-1.0 / _TAU)) * shr

    dm_col = jnp.transpose(dmin_h)
    iota_r = jax.lax.broadcasted_iota(jnp.int32, (nh, nh), 0)
    iota_c = jax.lax.broadcasted_iota(jnp.int32, (nh, nh), 1)
    smaller = (dm_col < dmin_h) | ((dm_col == dmin_h) & (iota_r < iota_c))
    rank = jnp.dot(jnp.ones((1, nh), jnp.float32),
                   smaller.astype(jnp.float32),
                   preferred_element_type=jnp.float32)

    qmeans = []
    for kq in kqs:
        sel = jnp.where(rank < kq, dmin_h, 0.0)
        qmeans.append(jnp.sum(sel, axis=1, keepdims=True) * (1.0 / kq))

    mean_dh = jnp.sum(dmin_h, axis=1, keepdims=True) * (1.0 / nh)
    min_dh = jnp.min(dmin_h, axis=1, keepdims=True)
    mean_wh = jnp.sum(w_h, axis=1, keepdims=True) * (1.0 / nh)
    mean_do = jnp.sum(dmin_o, axis=1, keepdims=True) * (1.0 / no)

    f_ref[...] = jnp.concatenate(
        [mean_dh, min_dh, qmeans[0], qmeans[1], qmeans[2],
         mean_wh, dir_mean, mean_do], axis=1)[None]


def _mlp_body(f_ref, w1_ref, b1_ref, w2_ref, b2_ref, out_ref):
    hid = jnp.maximum(
        jnp.dot(f_ref[...], w1_ref[...], preferred_element_type=jnp.float32)
        + b1_ref[...], 0.0)
    out_ref[...] = (
        jnp.dot(hid, w2_ref[...], preferred_element_type=jnp.float32)
        + b2_ref[...])


def kernel(human_bt_n3, object_bt_m3, s_h_bt_n, s_o_bt_m, W1, b1, W2, b2):
    B, T, Nh, _ = human_bt_n3.shape
    No = object_bt_m3.shape[2]
    BT = B * T
    h = human_bt_n3.reshape(BT, Nh, 3)
    o = object_bt_m3.reshape(BT, No, 3)
    hT = h.transpose(0, 2, 1)
    oT = o.transpose(0, 2, 1)
    shr = s_h_bt_n.reshape(BT, 1, Nh)
    kqs = tuple(int(max(1, round(q * Nh))) for q in (0.2, 0.5, 0.8))

    feats = pl.pallas_call(
        functools.partial(_feats_body, nh=Nh, no=No, kqs=kqs),
        grid=(BT,),
        in_specs=[
            pl.BlockSpec((1, No, 3), lambda i: (i, 0, 0)),
            pl.BlockSpec((1, 3, Nh), lambda i: (i, 0, 0)),
            pl.BlockSpec((1, 3, No), lambda i: (i, 0, 0)),
            pl.BlockSpec((1, 1, Nh), lambda i: (i, 0, 0)),
        ],
        out_specs=pl.BlockSpec((1, 1, 10), lambda i: (i, 0, 0)),
        out_shape=jax.ShapeDtypeStruct((BT, 1, 10), jnp.float32),
    )(o, hT, oT, shr)
    feats = feats.reshape(BT, 10)

    H = W1.shape[1]
    F = W2.shape[1]
    out = pl.pallas_call(
        _mlp_body,
        in_specs=[pl.BlockSpec(feats.shape, lambda: (0, 0)),
                  pl.BlockSpec(W1.shape, lambda: (0, 0)),
                  pl.BlockSpec((1, H), lambda: (0, 0)),
                  pl.BlockSpec(W2.shape, lambda: (0, 0)),
                  pl.BlockSpec((1, F), lambda: (0, 0))],
        out_specs=pl.BlockSpec((BT, F), lambda: (0, 0)),
        out_shape=jax.ShapeDtypeStruct((BT, F), jnp.float32),
    )(feats, W1, b1.reshape(1, H), W2, b2.reshape(1, F))
    return out.reshape(B, T, F)


# speedup vs baseline: 2.0564x; 1.0422x over previous
import functools

import jax
import jax.numpy as jnp
from jax.experimental import pallas as pl

_TAU = 0.05


def _feats_body(om2_ref, hT_ref, oT_ref, b2c_ref, a2r_ref, sh_ref, f_ref,
                *, nh, no, kqs):
    om2 = om2_ref[0]
    hT = hT_ref[0]
    oT = oT_ref[0]
    b2c = b2c_ref[0]
    a2r = a2r_ref[0]
    shr = sh_ref[0]

    gT2 = jnp.dot(om2, hT, preferred_element_type=jnp.float32)
    sqT = (b2c + a2r) + gT2

    sqmin_h = jnp.min(sqT, axis=0, keepdims=True)
    sqmin_o = jnp.min(sqT, axis=1, keepdims=True)
    dmin_h = jnp.sqrt(jnp.maximum(sqmin_h, 1e-12))
    dmin_o = jnp.sqrt(jnp.maximum(jnp.transpose(sqmin_o), 1e-12))

    eqf = (sqT == sqmin_h).astype(jnp.float32)
    o_nn = jnp.dot(oT, eqf, preferred_element_type=jnp.float32)
    cnt = jnp.dot(jnp.ones((1, no), jnp.float32), eqf,
                  preferred_element_type=jnp.float32)

    vecT = o_nn / cnt - hT
    nrm = jnp.sqrt(jnp.maximum(
        jnp.sum(vecT * vecT, axis=0, keepdims=True), 1e-6))
    dir_sum = jnp.sum(vecT / nrm, axis=1, keepdims=True)
    dir_mean = jnp.transpose(dir_sum) * (1.0 / nh)

    w_h = jnp.exp(dmin_h * (-1.0 / _TAU)) * shr

    dm_col = jnp.transpose(dmin_h)
    ones_row = jnp.ones((1, nh), jnp.float32)
    lt = (dm_col < dmin_h).astype(jnp.float32)
    le = (dm_col <= dmin_h).astype(jnp.float32)
    r1 = jnp.dot(ones_row, lt, preferred_element_type=jnp.float32)
    rle = jnp.dot(ones_row, le, preferred_element_type=jnp.float32)
    inv_e = 1.0 / (rle - r1)

    qmeans = []
    for kq in kqs:
        frac = jnp.clip((kq - r1) * inv_e, 0.0, 1.0)
        qmeans.append(
            jnp.sum(dmin_h * frac, axis=1, keepdims=True) * (1.0 / kq))

    mean_dh = jnp.sum(dmin_h, axis=1, keepdims=True) * (1.0 / nh)
    min_dh = jnp.min(dmin_h, axis=1, keepdims=True)
    mean_wh = jnp.sum(w_h, axis=1, keepdims=True) * (1.0 / nh)
    mean_do = jnp.sum(dmin_o, axis=1, keepdims=True) * (1.0 / no)

    f_ref[...] = jnp.concatenate(
        [mean_dh, min_dh, qmeans[0], qmeans[1], qmeans[2],
         mean_wh, dir_mean, mean_do], axis=1)[None]


def _mlp_body(f_ref, w1_ref, b1_ref, w2_ref, b2_ref, out_ref):
    hid = jnp.maximum(
        jnp.dot(f_ref[...], w1_ref[...], preferred_element_type=jnp.float32)
        + b1_ref[...], 0.0)
    out_ref[...] = (
        jnp.dot(hid, w2_ref[...], preferred_element_type=jnp.float32)
        + b2_ref[...])


def kernel(human_bt_n3, object_bt_m3, s_h_bt_n, s_o_bt_m, W1, b1, W2, b2):
    B, T, Nh, _ = human_bt_n3.shape
    No = object_bt_m3.shape[2]
    BT = B * T
    h = human_bt_n3.reshape(BT, Nh, 3)
    o = object_bt_m3.reshape(BT, No, 3)
    hT = h.transpose(0, 2, 1)
    oT = o.transpose(0, 2, 1)
    om2 = -2.0 * o
    a2r = jnp.sum(h * h, axis=2)[:, None, :]
    b2c = jnp.sum(o * o, axis=2)[:, :, None]
    shr = s_h_bt_n.reshape(BT, 1, Nh)
    kqs = tuple(int(max(1, round(q * Nh))) for q in (0.2, 0.5, 0.8))

    feats = pl.pallas_call(
        functools.partial(_feats_body, nh=Nh, no=No, kqs=kqs),
        grid=(BT,),
        in_specs=[
            pl.BlockSpec((1, No, 3), lambda i: (i, 0, 0)),
            pl.BlockSpec((1, 3, Nh), lambda i: (i, 0, 0)),
            pl.BlockSpec((1, 3, No), lambda i: (i, 0, 0)),
            pl.BlockSpec((1, No, 1), lambda i: (i, 0, 0)),
            pl.BlockSpec((1, 1, Nh), lambda i: (i, 0, 0)),
            pl.BlockSpec((1, 1, Nh), lambda i: (i, 0, 0)),
        ],
        out_specs=pl.BlockSpec((1, 1, 10), lambda i: (i, 0, 0)),
        out_shape=jax.ShapeDtypeStruct((BT, 1, 10), jnp.float32),
    )(om2, hT, oT, b2c, a2r, shr)
    feats = feats.reshape(BT, 10)

    H = W1.shape[1]
    F = W2.shape[1]
    out = pl.pallas_call(
        _mlp_body,
        in_specs=[pl.BlockSpec(feats.shape, lambda: (0, 0)),
                  pl.BlockSpec(W1.shape, lambda: (0, 0)),
                  pl.BlockSpec((1, H), lambda: (0, 0)),
                  pl.BlockSpec(W2.shape, lambda: (0, 0)),
                  pl.BlockSpec((1, F), lambda: (0, 0))],
        out_specs=pl.BlockSpec((BT, F), lambda: (0, 0)),
        out_shape=jax.ShapeDtypeStruct((BT, F), jnp.float32),
    )(feats, W1, b1.reshape(1, H), W2, b2.reshape(1, F))
    return out.reshape(B, T, F)
